# baseline (speedup 1.0000x reference)
import functools

import jax
import jax.numpy as jnp
from jax import lax
from jax.experimental import pallas as pl
from jax.experimental.pallas import tpu as pltpu
from jax.experimental.pallas import tpu_sc as plsc

_LANES = 16
_CHUNK_ROWS = 32
_UNROLL = 8


def _make_sc_kernel(batch, seq_len, d_model, pos_rows):
    info = plsc.get_sparse_core_info()
    nc, ns = info.num_cores, info.num_subcores
    nw = nc * ns
    total_rows = batch * seq_len
    rows_per_w = total_rows // nw
    c = _CHUNK_ROWS
    chunk_elems = c * d_model
    pos_elems = pos_rows * d_model
    nchunks = rows_per_w // c
    mesh = plsc.VectorSubcoreMesh(core_axis_name="c", subcore_axis_name="s")

    @functools.partial(
        pl.kernel,
        mesh=mesh,
        out_type=jax.ShapeDtypeStruct((total_rows * d_model,), jnp.float32),
        scratch_types=[
            pltpu.VMEM((chunk_elems,), jnp.float32),
            pltpu.VMEM((chunk_elems,), jnp.float32),
        ],
    )
    def sc_add(x_hbm, pos_hbm, out_hbm, x_v, pos_v):
        wid = lax.axis_index("s") * nc + lax.axis_index("c")
        base_elem = wid * (rows_per_w * d_model)

        def chunk_body(j, carry):
            e0 = base_elem + j * chunk_elems
            p0 = lax.rem(e0, pos_elems)
            pltpu.sync_copy(x_hbm.at[pl.ds(e0, chunk_elems)], x_v)
            pltpu.sync_copy(pos_hbm.at[pl.ds(p0, chunk_elems)], pos_v)

            @plsc.parallel_loop(0, chunk_elems // _LANES, step=1, unroll=_UNROLL)
            def add_body(i):
                s = pl.ds(i * _LANES, _LANES)
                x_v[s] = x_v[s] + pos_v[s]
            pltpu.sync_copy(x_v, out_hbm.at[pl.ds(e0, chunk_elems)])
            return carry

        lax.fori_loop(0, nchunks, chunk_body, 0)

    return sc_add


def kernel(x, pos_weight):
    batch, seq_len, d_model = x.shape
    pos = pos_weight[:seq_len]
    sc = _make_sc_kernel(batch, seq_len, d_model, seq_len)
    out = sc(x.reshape(-1), pos.reshape(-1))
    return out.reshape(x.shape)


# speedup vs baseline: 1.8984x; 1.8984x over previous
import functools

import jax
import jax.numpy as jnp
from jax import lax
from jax.experimental import pallas as pl
from jax.experimental.pallas import tpu as pltpu
from jax.experimental.pallas import tpu_sc as plsc

_LANES = 16
_CHUNK_ROWS = 32
_UNROLL = 8


def _make_sc_kernel(batch, seq_len, d_model):
    info = plsc.get_sparse_core_info()
    nc, ns = info.num_cores, info.num_subcores
    nw = nc * ns
    total_rows = batch * seq_len
    rows_per_w = total_rows // nw
    w_per_batch = seq_len // rows_per_w
    c = _CHUNK_ROWS
    vecs_per_row = d_model // _LANES
    nchunks = rows_per_w // c
    mesh = plsc.VectorSubcoreMesh(core_axis_name="c", subcore_axis_name="s")

    @functools.partial(
        pl.kernel,
        mesh=mesh,
        out_type=jax.ShapeDtypeStruct((batch, seq_len, d_model), jnp.float32),
        scratch_types=[
            pltpu.VMEM((c, d_model), jnp.float32),
            pltpu.VMEM((c, d_model), jnp.float32),
        ],
    )
    def sc_add(x_hbm, pos_hbm, out_hbm, x_v, pos_v):
        wid = lax.axis_index("s") * nc + lax.axis_index("c")
        b = wid // w_per_batch
        s0 = (wid % w_per_batch) * rows_per_w

        def chunk_body(j, carry):
            s = s0 + j * c
            pltpu.sync_copy(x_hbm.at[b, pl.ds(s, c)], x_v)
            pltpu.sync_copy(pos_hbm.at[pl.ds(s, c)], pos_v)

            @plsc.parallel_loop(0, c * vecs_per_row, step=1, unroll=_UNROLL)
            def add_body(i):
                r = i // vecs_per_row
                k = lax.rem(i, vecs_per_row) * _LANES
                sl = pl.ds(k, _LANES)
                x_v[r, sl] = x_v[r, sl] + pos_v[r, sl]

            pltpu.sync_copy(x_v, out_hbm.at[b, pl.ds(s, c)])
            return carry

        lax.fori_loop(0, nchunks, chunk_body, 0)

    return sc_add


def kernel(x, pos_weight):
    batch, seq_len, d_model = x.shape
    sc = _make_sc_kernel(batch, seq_len, d_model)
    return sc(x, pos_weight[:seq_len])


# speedup vs baseline: 2.7304x; 1.4383x over previous
import functools

import jax
import jax.numpy as jnp
from jax import lax
from jax.experimental import pallas as pl
from jax.experimental.pallas import tpu as pltpu
from jax.experimental.pallas import tpu_sc as plsc

_LANES = 16
_CHUNK_ROWS = 16
_NBUF = 2
_UNROLL = 8


def _make_sc_kernel(batch, seq_len, d_model):
    info = plsc.get_sparse_core_info()
    nc, ns = info.num_cores, info.num_subcores
    nw = nc * ns
    total_rows = batch * seq_len
    rows_per_w = total_rows // nw
    w_per_batch = seq_len // rows_per_w
    c = _CHUNK_ROWS
    vecs_per_row = d_model // _LANES
    nchunks = rows_per_w // c
    mesh = plsc.VectorSubcoreMesh(core_axis_name="c", subcore_axis_name="s")

    buf = lambda: pltpu.VMEM((c, d_model), jnp.float32)
    sem = lambda: pltpu.SemaphoreType.DMA

    @functools.partial(
        pl.kernel,
        mesh=mesh,
        out_type=jax.ShapeDtypeStruct((batch, seq_len, d_model), jnp.float32),
        scratch_types=(
            [buf() for _ in range(_NBUF)]
            + [buf() for _ in range(_NBUF)]
            + [buf() for _ in range(_NBUF)]
            + [sem() for _ in range(3 * _NBUF)]
        ),
    )
    def sc_add(x_hbm, pos_hbm, out_hbm, *scratch):
        x_bufs = scratch[0:_NBUF]
        p_bufs = scratch[_NBUF:2 * _NBUF]
        o_bufs = scratch[2 * _NBUF:3 * _NBUF]
        sems = scratch[3 * _NBUF:]
        sx = sems[0:_NBUF]
        sp = sems[_NBUF:2 * _NBUF]
        so = sems[2 * _NBUF:]

        wid = lax.axis_index("s") * nc + lax.axis_index("c")
        b = wid // w_per_batch
        s0 = (wid % w_per_batch) * rows_per_w

        def x_cp(j, u):
            s = s0 + j * c
            return pltpu.make_async_copy(x_hbm.at[b, pl.ds(s, c)], x_bufs[u], sx[u])

        def p_cp(j, u):
            s = s0 + j * c
            return pltpu.make_async_copy(pos_hbm.at[pl.ds(s, c)], p_bufs[u], sp[u])

        def o_cp(j, u):
            s = s0 + j * c
            return pltpu.make_async_copy(o_bufs[u], out_hbm.at[b, pl.ds(s, c)], so[u])

        for u in range(_NBUF):
            x_cp(u, u).start()
            p_cp(u, u).start()

        def round_body(t, carry):
            for u in range(_NBUF):
                j = t * _NBUF + u
                x_cp(j, u).wait()
                p_cp(j, u).wait()

                @pl.when(t > 0)
                def _():
                    o_cp(j - _NBUF, u).wait()

                x_v, pos_v, o_v = x_bufs[u], p_bufs[u], o_bufs[u]

                @plsc.parallel_loop(0, c * vecs_per_row, step=1, unroll=_UNROLL)
                def add_body(i):
                    r = i // vecs_per_row
                    k = lax.rem(i, vecs_per_row) * _LANES
                    sl = pl.ds(k, _LANES)
                    o_v[r, sl] = x_v[r, sl] + pos_v[r, sl]

                o_cp(j, u).start()

                @pl.when(j + _NBUF < nchunks)
                def _():
                    x_cp(j + _NBUF, u).start()
                    p_cp(j + _NBUF, u).start()
            return carry

        lax.fori_loop(0, nchunks // _NBUF, round_body, 0)

        for u in range(_NBUF):
            o_cp(nchunks - _NBUF + u, u).wait()

    return sc_add


def kernel(x, pos_weight):
    batch, seq_len, d_model = x.shape
    sc = _make_sc_kernel(batch, seq_len, d_model)
    return sc(x, pos_weight[:seq_len])
